# baseline (speedup 1.0000x reference)
import functools

import jax
import jax.numpy as jnp
from jax import lax
from jax.experimental import pallas as pl
from jax.experimental.pallas import tpu as pltpu
from jax.experimental.pallas import tpu_sc as plsc

N_SC = 2
N_SUB = 16
LANES = 16
CHUNK = 64
DHALF = 64


def _agg_body(nodes2, src_hbm, dst_hbm, f_hbm, zeros2d, zeros1d,
              s1o, spno, dego,
              srcv, dstv, dst2v, fv, xbuf, onesbuf,
              acc1, accpn, accdeg, gsem, apnsem):
    c = lax.axis_index("c")
    s = lax.axis_index("s")
    n_nodes = acc1.shape[0]
    rows_base = (n_nodes // N_SUB) & ~7
    rows_last = n_nodes - rows_base * (N_SUB - 1)
    n_edges = src_hbm.shape[0]
    n_chunks = n_edges // CHUNK
    base_chunks = n_chunks // N_SUB
    extra = n_chunks - base_chunks * N_SUB

    def fill_ones(i, _):
        onesbuf[pl.ds(i * LANES, LANES)] = jnp.ones((LANES,), jnp.float32)
        return 0
    lax.fori_loop(0, CHUNK // LANES, fill_ones, 0)

    r0 = s * rows_base

    def zero_accs(nrows):
        z = zeros2d.at[pl.ds(0, nrows)]
        pltpu.sync_copy(z, acc1.at[pl.ds(r0, nrows)])
        pltpu.sync_copy(z, accpn.at[pl.ds(r0, nrows)])
        pltpu.sync_copy(z, accpn.at[pl.ds(n_nodes + r0, nrows)])

    @pl.when(s < N_SUB - 1)
    def _():
        zero_accs(rows_base)

    @pl.when(s == N_SUB - 1)
    def _():
        zero_accs(rows_last)

    @pl.when(jnp.logical_and(c == 0, s == 0))
    def _():
        pltpu.sync_copy(zeros1d, accdeg)

    plsc.subcore_barrier()

    cN = jnp.full((LANES,), c * n_nodes, jnp.int32)
    nN = jnp.full((LANES,), n_nodes, jnp.int32)
    g_lo = s * base_chunks + jnp.minimum(s, extra)
    g_hi = g_lo + base_chunks + jnp.where(s < extra, 1, 0)

    def load_and_launch(g, slot):
        base = g * CHUNK
        pltpu.sync_copy(src_hbm.at[pl.ds(base, CHUNK)], srcv.at[slot, 0])
        pltpu.sync_copy(dst_hbm.at[pl.ds(base, CHUNK)], dstv.at[slot, 0])
        pltpu.sync_copy(f_hbm.at[pl.ds(base, CHUNK)], fv.at[slot, 0])

        def adjust(i, _):
            sl = pl.ds(i * LANES, LANES)
            srcv[slot, 0, sl] = srcv[slot, 0, sl] + cN
            dst2v[slot, 0, sl] = dstv[slot, 0, sl] + jnp.where(
                fv[slot, 0, sl] < 0.0, nN, 0)
            return 0
        lax.fori_loop(0, CHUNK // LANES, adjust, 0)
        pltpu.async_copy(nodes2.at[srcv.at[slot, 0]], xbuf.at[slot],
                         gsem.at[slot])

    def drain_apn(slot):
        pltpu.make_async_copy(
            xbuf.at[slot], accpn.at[dst2v.at[slot, 0]], apnsem.at[slot]).wait()

    load_and_launch(g_lo, 0)

    def chunk_body(g, _):
        b = lax.rem(g - g_lo, 2)
        nb = 1 - b

        @pl.when(g + 1 < g_hi)
        def _():
            @pl.when(g - 1 >= g_lo)
            def _():
                drain_apn(nb)
            load_and_launch(g + 1, nb)

        pltpu.make_async_copy(
            nodes2.at[srcv.at[b, 0]], xbuf.at[b], gsem.at[b]).wait()
        pltpu.sync_copy(xbuf.at[b], acc1.at[dstv.at[b, 0]], add=True)

        @pl.when(c == 0)
        def _():
            pltpu.sync_copy(onesbuf, accdeg.at[dstv.at[b, 0]], add=True)

        def group_body(g16, _):
            afvec = jnp.abs(fv[b, 0, pl.ds(g16 * LANES, LANES)])
            for j in range(LANES):
                e = g16 * LANES + j
                afb = jnp.full((LANES,), afvec[j], jnp.float32)
                for cc in range(DHALF // LANES):
                    sl = pl.ds(cc * LANES, LANES)
                    xbuf[b, e, sl] = afb * xbuf[b, e, sl]
            return 0
        lax.fori_loop(0, CHUNK // LANES, group_body, 0)

        pltpu.async_copy(
            xbuf.at[b], accpn.at[dst2v.at[b, 0]], apnsem.at[b], add=True)
        return 0

    lax.fori_loop(g_lo, g_hi, chunk_body, 0)

    lb = lax.rem(g_hi - 1 - g_lo, 2)
    drain_apn(lb)

    @pl.when(g_hi - g_lo >= 2)
    def _():
        drain_apn(1 - lb)

    plsc.subcore_barrier()

    def copy_out(nrows):
        pltpu.sync_copy(acc1.at[pl.ds(r0, nrows)], s1o.at[c, pl.ds(r0, nrows)])
        pltpu.sync_copy(accpn.at[pl.ds(r0, nrows)], spno.at[c, pl.ds(r0, nrows)])
        pltpu.sync_copy(accpn.at[pl.ds(n_nodes + r0, nrows)],
                        spno.at[c, pl.ds(n_nodes + r0, nrows)])

    @pl.when(s < N_SUB - 1)
    def _():
        copy_out(rows_base)

    @pl.when(s == N_SUB - 1)
    def _():
        copy_out(rows_last)

    @pl.when(jnp.logical_and(c == 0, s == 0))
    def _():
        pltpu.sync_copy(accdeg, dego)


def _sc_aggregate(nodes2, src, dst, fw, n_nodes):
    out_type = (
        jax.ShapeDtypeStruct((N_SC, n_nodes, DHALF), jnp.float32),
        jax.ShapeDtypeStruct((N_SC, 2 * n_nodes, DHALF), jnp.float32),
        jax.ShapeDtypeStruct((n_nodes,), jnp.float32),
    )
    scratch = [
        pltpu.VMEM((2, 1, CHUNK), jnp.int32),
        pltpu.VMEM((2, 1, CHUNK), jnp.int32),
        pltpu.VMEM((2, 1, CHUNK), jnp.int32),
        pltpu.VMEM((2, 1, CHUNK), jnp.float32),
        pltpu.VMEM((2, CHUNK, DHALF), jnp.float32),
        pltpu.VMEM((CHUNK,), jnp.float32),
        pltpu.VMEM_SHARED((n_nodes, DHALF), jnp.float32),
        pltpu.VMEM_SHARED((2 * n_nodes, DHALF), jnp.float32),
        pltpu.VMEM_SHARED((n_nodes,), jnp.float32),
        pltpu.SemaphoreType.DMA((2,)),
        pltpu.SemaphoreType.DMA((2,)),
    ]
    rows_base = (n_nodes // N_SUB) & ~7
    rows_last = n_nodes - rows_base * (N_SUB - 1)
    zeros2d = jnp.zeros((rows_last, DHALF), jnp.float32)
    zeros1d = jnp.zeros((n_nodes,), jnp.float32)
    run = pl.kernel(
        _agg_body,
        out_type=out_type,
        mesh=plsc.VectorSubcoreMesh(core_axis_name="c", subcore_axis_name="s"),
        scratch_types=scratch,
        compiler_params=pltpu.CompilerParams(use_tc_tiling_on_sc=False),
    )
    return run(nodes2, src, dst, fw, zeros2d, zeros1d)


def _dense_pass1(node_ref, s1a_ref, s1b_ref, pa_ref, pb_ref, na_ref, nb_ref,
                 deg_ref, fdig_ref, normn_ref, w_ref, b_ref,
                 h_ref, psum_ref, psumsq_ref):
    nf = node_ref[...]
    deg = jnp.maximum(deg_ref[...], 1.0)
    w = w_ref[...]
    dot = functools.partial(jnp.dot, preferred_element_type=jnp.float32)
    s2a = pa_ref[...] + na_ref[...]
    s2b = pb_ref[...] + nb_ref[...]
    s3a = pa_ref[...] - na_ref[...]
    s3b = pb_ref[...] - nb_ref[...]
    h_mean = dot(s1a_ref[...], w[128:192]) + dot(s1b_ref[...], w[192:256])
    h_av = dot(s2a, w[256:320]) + dot(s2b, w[320:384])
    h_dx = dot(s3a, w[384:448]) + dot(s3b, w[448:512])
    h = (dot(nf, w[0:128]) + h_mean / deg + h_av + h_dx
         - fdig_ref[...] * dot(nf, w[384:512]) + b_ref[...])
    h = h * normn_ref[...]
    h_ref[...] = h
    psum_ref[...] = jnp.sum(h, axis=0, keepdims=True)[None]
    psumsq_ref[...] = jnp.sum(h * h, axis=0, keepdims=True)[None]


def _dense_pass2(h_ref, psum_ref, psumsq_ref, gamma_ref, beta_ref, out_ref,
                 *, n_nodes):
    mu = jnp.sum(psum_ref[...], axis=0) / n_nodes
    var = jnp.sum(psumsq_ref[...], axis=0) / n_nodes - mu * mu
    h = (h_ref[...] - mu) * lax.rsqrt(var + 1e-5) * gamma_ref[...] + beta_ref[...]
    out_ref[...] = jnp.maximum(h, 0.0)


def kernel(node_fts, edge_fts, edge_index, F_norm_edge, F_dig, node_deg_vec, norm_n, W, b, gamma, beta):
    n_nodes = node_fts.shape[0]
    src = edge_index[0]
    dst = edge_index[1]
    fw = F_norm_edge[:, 0]
    nodes2 = jnp.concatenate([node_fts[:, :DHALF], node_fts[:, DHALF:]], axis=0)

    s1o, spno, deg = _sc_aggregate(nodes2, src, dst, fw, n_nodes)

    grid = 10
    blk = n_nodes // grid
    row_full = lambda width: pl.BlockSpec((blk, width), lambda i: (i, 0))
    whole = lambda shape: pl.BlockSpec(shape, lambda i: (0,) * len(shape))
    stat_spec = pl.BlockSpec((1, 1, 128), lambda i: (i, 0, 0))
    h, psum, psumsq = pl.pallas_call(
        _dense_pass1,
        grid=(grid,),
        in_specs=[
            row_full(128), row_full(64), row_full(64), row_full(64),
            row_full(64), row_full(64), row_full(64),
            row_full(1), row_full(1), row_full(1),
            whole((512, 128)), whole((1, 128)),
        ],
        out_specs=[row_full(128), stat_spec, stat_spec],
        out_shape=[
            jax.ShapeDtypeStruct((n_nodes, 128), jnp.float32),
            jax.ShapeDtypeStruct((grid, 1, 128), jnp.float32),
            jax.ShapeDtypeStruct((grid, 1, 128), jnp.float32),
        ],
    )(node_fts, s1o[0], s1o[1],
      spno[0, :n_nodes], spno[1, :n_nodes], spno[0, n_nodes:], spno[1, n_nodes:],
      deg[:, None], F_dig, norm_n, W, b[None, :])

    return pl.pallas_call(
        functools.partial(_dense_pass2, n_nodes=n_nodes),
        grid=(grid,),
        in_specs=[
            row_full(128),
            whole((grid, 1, 128)), whole((grid, 1, 128)),
            whole((1, 128)), whole((1, 128)),
        ],
        out_specs=row_full(128),
        out_shape=jax.ShapeDtypeStruct((n_nodes, 128), jnp.float32),
    )(h, psum, psumsq, gamma[None, :], beta[None, :])


# speedup vs baseline: 2.0802x; 2.0802x over previous
import functools

import jax
import jax.numpy as jnp
from jax import lax
from jax.experimental import pallas as pl
from jax.experimental.pallas import tpu as pltpu
from jax.experimental.pallas import tpu_sc as plsc

N_SC = 2
N_SUB = 16
LANES = 16
CHUNK = 128
HALF = CHUNK // 2
DHALF = 64


def _agg_body(nodes2, src_hbm, dst_hbm, f_hbm, zeros2d, zeros1d,
              s1o, spno, dego,
              srcv, fv, dstva, dstvb, dst2a, dst2b, xbuf, onesbuf,
              acc1, accpn, accdeg, isem, gsem, a1sem, dsem, apnsem):
    c = lax.axis_index("c")
    s = lax.axis_index("s")
    n_nodes = acc1.shape[0]
    rows_base = (n_nodes // N_SUB) & ~7
    rows_last = n_nodes - rows_base * (N_SUB - 1)
    n_edges = src_hbm.shape[0]
    n_chunks = n_edges // CHUNK
    base_chunks = n_chunks // N_SUB
    extra = n_chunks - base_chunks * N_SUB

    def fill_ones(i, _):
        onesbuf[pl.ds(i * LANES, LANES)] = jnp.ones((LANES,), jnp.float32)
        return 0
    lax.fori_loop(0, HALF // LANES, fill_ones, 0)

    r0 = s * rows_base

    def zero_accs(nrows):
        z = zeros2d.at[pl.ds(0, nrows)]
        pltpu.sync_copy(z, acc1.at[pl.ds(r0, nrows)])
        pltpu.sync_copy(z, accpn.at[pl.ds(r0, nrows)])
        pltpu.sync_copy(z, accpn.at[pl.ds(n_nodes + r0, nrows)])

    @pl.when(s < N_SUB - 1)
    def _():
        zero_accs(rows_base)

    @pl.when(s == N_SUB - 1)
    def _():
        zero_accs(rows_last)

    @pl.when(jnp.logical_and(c == 0, s == 0))
    def _():
        pltpu.sync_copy(zeros1d, accdeg)

    plsc.subcore_barrier()

    cN = jnp.full((LANES,), c * n_nodes, jnp.int32)
    nN = jnp.full((LANES,), n_nodes, jnp.int32)
    g_lo = s * base_chunks + jnp.minimum(s, extra)
    g_hi = g_lo + base_chunks + jnp.where(s < extra, 1, 0)

    def drain_apn():
        pltpu.make_async_copy(
            xbuf.at[pl.ds(0, HALF)], accpn.at[dst2a], apnsem).wait()
        pltpu.make_async_copy(
            xbuf.at[pl.ds(HALF, HALF)], accpn.at[dst2b], apnsem).wait()

    def scale_half(h):
        def group_body(g16, _):
            afvec = jnp.abs(fv[pl.ds(g16 * LANES, LANES)])
            for j in range(LANES):
                e = g16 * LANES + j
                afb = jnp.full((LANES,), afvec[j], jnp.float32)
                for cc in range(DHALF // LANES):
                    sl = pl.ds(cc * LANES, LANES)
                    xbuf[e, sl] = afb * xbuf[e, sl]
            return 0
        lo = h * (HALF // LANES)
        lax.fori_loop(lo, lo + HALF // LANES, group_body, 0)

    def chunk_body(g, _):
        base = g * CHUNK
        pltpu.async_copy(src_hbm.at[pl.ds(base, CHUNK)], srcv, isem)
        pltpu.async_copy(f_hbm.at[pl.ds(base, CHUNK)], fv, isem)
        pltpu.async_copy(dst_hbm.at[pl.ds(base, HALF)], dstva, isem)
        pltpu.async_copy(dst_hbm.at[pl.ds(base + HALF, HALF)], dstvb, isem)

        @pl.when(g > g_lo)
        def _():
            drain_apn()

        pltpu.make_async_copy(src_hbm.at[pl.ds(base, CHUNK)], srcv, isem).wait()
        pltpu.make_async_copy(f_hbm.at[pl.ds(base, CHUNK)], fv, isem).wait()
        pltpu.make_async_copy(dst_hbm.at[pl.ds(base, HALF)], dstva, isem).wait()
        pltpu.make_async_copy(dst_hbm.at[pl.ds(base + HALF, HALF)], dstvb,
                              isem).wait()

        def adjust(i, _):
            sl = pl.ds(i * LANES, LANES)
            srcv[sl] = srcv[sl] + cN
            return 0
        lax.fori_loop(0, CHUNK // LANES, adjust, 0)

        def adjust2(i, _):
            sl = pl.ds(i * LANES, LANES)
            slb = pl.ds(HALF + i * LANES, LANES)
            dst2a[sl] = dstva[sl] + jnp.where(fv[sl] < 0.0, nN, 0)
            dst2b[sl] = dstvb[sl] + jnp.where(fv[slb] < 0.0, nN, 0)
            return 0
        lax.fori_loop(0, HALF // LANES, adjust2, 0)

        pltpu.async_copy(nodes2.at[srcv], xbuf, gsem).wait()

        xa = xbuf.at[pl.ds(0, HALF)]
        xb = xbuf.at[pl.ds(HALF, HALF)]
        pltpu.async_copy(xa, acc1.at[dstva], a1sem)

        @pl.when(c == 0)
        def _():
            pltpu.async_copy(onesbuf, accdeg.at[dstva], dsem)
        scale_half(1)
        pltpu.make_async_copy(xa, acc1.at[dstva], a1sem).wait()
        pltpu.async_copy(xb, acc1.at[dstvb], a1sem)

        @pl.when(c == 0)
        def _():
            pltpu.async_copy(onesbuf, accdeg.at[dstvb], dsem)
        scale_half(0)
        pltpu.make_async_copy(xb, acc1.at[dstvb], a1sem).wait()

        @pl.when(c == 0)
        def _():
            pltpu.make_async_copy(onesbuf, accdeg.at[dstva], dsem).wait()
            pltpu.make_async_copy(onesbuf, accdeg.at[dstvb], dsem).wait()

        pltpu.async_copy(xa, accpn.at[dst2a], apnsem, add=True)
        pltpu.async_copy(xb, accpn.at[dst2b], apnsem, add=True)
        return 0

    lax.fori_loop(g_lo, g_hi, chunk_body, 0)
    drain_apn()

    plsc.subcore_barrier()

    def copy_out(nrows):
        pltpu.sync_copy(acc1.at[pl.ds(r0, nrows)], s1o.at[c, pl.ds(r0, nrows)])
        pltpu.sync_copy(accpn.at[pl.ds(r0, nrows)], spno.at[c, pl.ds(r0, nrows)])
        pltpu.sync_copy(accpn.at[pl.ds(n_nodes + r0, nrows)],
                        spno.at[c, pl.ds(n_nodes + r0, nrows)])

    @pl.when(s < N_SUB - 1)
    def _():
        copy_out(rows_base)

    @pl.when(s == N_SUB - 1)
    def _():
        copy_out(rows_last)

    @pl.when(jnp.logical_and(c == 0, s == 0))
    def _():
        pltpu.sync_copy(accdeg, dego)


def _sc_aggregate(nodes2, src, dst, fw, n_nodes):
    out_type = (
        jax.ShapeDtypeStruct((N_SC, n_nodes, DHALF), jnp.float32),
        jax.ShapeDtypeStruct((N_SC, 2 * n_nodes, DHALF), jnp.float32),
        jax.ShapeDtypeStruct((n_nodes,), jnp.float32),
    )
    scratch = [
        pltpu.VMEM((CHUNK,), jnp.int32),
        pltpu.VMEM((CHUNK,), jnp.float32),
        pltpu.VMEM((HALF,), jnp.int32),
        pltpu.VMEM((HALF,), jnp.int32),
        pltpu.VMEM((HALF,), jnp.int32),
        pltpu.VMEM((HALF,), jnp.int32),
        pltpu.VMEM((CHUNK, DHALF), jnp.float32),
        pltpu.VMEM((HALF,), jnp.float32),
        pltpu.VMEM_SHARED((n_nodes, DHALF), jnp.float32),
        pltpu.VMEM_SHARED((2 * n_nodes, DHALF), jnp.float32),
        pltpu.VMEM_SHARED((n_nodes,), jnp.float32),
        pltpu.SemaphoreType.DMA,
        pltpu.SemaphoreType.DMA,
        pltpu.SemaphoreType.DMA,
        pltpu.SemaphoreType.DMA,
        pltpu.SemaphoreType.DMA,
    ]
    rows_base = (n_nodes // N_SUB) & ~7
    rows_last = n_nodes - rows_base * (N_SUB - 1)
    zeros2d = jnp.zeros((rows_last, DHALF), jnp.float32)
    zeros1d = jnp.zeros((n_nodes,), jnp.float32)
    run = pl.kernel(
        _agg_body,
        out_type=out_type,
        mesh=plsc.VectorSubcoreMesh(core_axis_name="c", subcore_axis_name="s"),
        scratch_types=scratch,
        compiler_params=pltpu.CompilerParams(use_tc_tiling_on_sc=False),
    )
    return run(nodes2, src, dst, fw, zeros2d, zeros1d)


def _dense_pass1(node_ref, s1a_ref, s1b_ref, pa_ref, pb_ref, na_ref, nb_ref,
                 deg_ref, fdig_ref, normn_ref, w_ref, b_ref,
                 h_ref, psum_ref, psumsq_ref):
    nf = node_ref[...]
    deg = jnp.maximum(deg_ref[...], 1.0)
    w = w_ref[...]
    dot = functools.partial(jnp.dot, preferred_element_type=jnp.float32)
    s2a = pa_ref[...] + na_ref[...]
    s2b = pb_ref[...] + nb_ref[...]
    s3a = pa_ref[...] - na_ref[...]
    s3b = pb_ref[...] - nb_ref[...]
    h_mean = dot(s1a_ref[...], w[128:192]) + dot(s1b_ref[...], w[192:256])
    h_av = dot(s2a, w[256:320]) + dot(s2b, w[320:384])
    h_dx = dot(s3a, w[384:448]) + dot(s3b, w[448:512])
    h = (dot(nf, w[0:128]) + h_mean / deg + h_av + h_dx
         - fdig_ref[...] * dot(nf, w[384:512]) + b_ref[...])
    h = h * normn_ref[...]
    h_ref[...] = h
    psum_ref[...] = jnp.sum(h, axis=0, keepdims=True)[None]
    psumsq_ref[...] = jnp.sum(h * h, axis=0, keepdims=True)[None]


def _dense_pass2(h_ref, psum_ref, psumsq_ref, gamma_ref, beta_ref, out_ref,
                 *, n_nodes):
    mu = jnp.sum(psum_ref[...], axis=0) / n_nodes
    var = jnp.sum(psumsq_ref[...], axis=0) / n_nodes - mu * mu
    h = (h_ref[...] - mu) * lax.rsqrt(var + 1e-5) * gamma_ref[...] + beta_ref[...]
    out_ref[...] = jnp.maximum(h, 0.0)


def kernel(node_fts, edge_fts, edge_index, F_norm_edge, F_dig, node_deg_vec, norm_n, W, b, gamma, beta):
    n_nodes = node_fts.shape[0]
    src = edge_index[0]
    dst = edge_index[1]
    fw = F_norm_edge[:, 0]
    nodes2 = jnp.concatenate([node_fts[:, :DHALF], node_fts[:, DHALF:]], axis=0)

    s1o, spno, deg = _sc_aggregate(nodes2, src, dst, fw, n_nodes)

    grid = 10
    blk = n_nodes // grid
    row_full = lambda width: pl.BlockSpec((blk, width), lambda i: (i, 0))
    whole = lambda shape: pl.BlockSpec(shape, lambda i: (0,) * len(shape))
    stat_spec = pl.BlockSpec((1, 1, 128), lambda i: (i, 0, 0))
    h, psum, psumsq = pl.pallas_call(
        _dense_pass1,
        grid=(grid,),
        in_specs=[
            row_full(128), row_full(64), row_full(64), row_full(64),
            row_full(64), row_full(64), row_full(64),
            row_full(1), row_full(1), row_full(1),
            whole((512, 128)), whole((1, 128)),
        ],
        out_specs=[row_full(128), stat_spec, stat_spec],
        out_shape=[
            jax.ShapeDtypeStruct((n_nodes, 128), jnp.float32),
            jax.ShapeDtypeStruct((grid, 1, 128), jnp.float32),
            jax.ShapeDtypeStruct((grid, 1, 128), jnp.float32),
        ],
    )(node_fts, s1o[0], s1o[1],
      spno[0, :n_nodes], spno[1, :n_nodes], spno[0, n_nodes:], spno[1, n_nodes:],
      deg[:, None], F_dig, norm_n, W, b[None, :])

    return pl.pallas_call(
        functools.partial(_dense_pass2, n_nodes=n_nodes),
        grid=(grid,),
        in_specs=[
            row_full(128),
            whole((grid, 1, 128)), whole((grid, 1, 128)),
            whole((1, 128)), whole((1, 128)),
        ],
        out_specs=row_full(128),
        out_shape=jax.ShapeDtypeStruct((n_nodes, 128), jnp.float32),
    )(h, psum, psumsq, gamma[None, :], beta[None, :])
